# Initial kernel scaffold; baseline (speedup 1.0000x reference)
import jax
import jax.numpy as jnp
from jax.experimental import pallas as pl


def kernel(x, edge_index, batch, w1_rel, b1, w1_root, w2_rel, b2, w2_root, w3_rel, b3, w3_root, wp_rel, bp, wp_root, lin1_w, lin1_b, lin2_w, lin2_b):
    raise NotImplementedError("write your pallas kernel here")


# speedup vs baseline: 1.1405x; 1.1405x over previous
import jax
import jax.numpy as jnp
from jax.experimental import pallas as pl


def _mm_body(a_ref, b_ref, o_ref):
    o_ref[...] = jnp.dot(a_ref[...], b_ref[...],
                         preferred_element_type=jnp.float32)


def _mm(a, b, bm=256):
    M, K = a.shape
    _, Nn = b.shape
    return pl.pallas_call(
        _mm_body,
        grid=(M // bm,),
        in_specs=[pl.BlockSpec((bm, K), lambda i: (i, 0)),
                  pl.BlockSpec((K, Nn), lambda i: (0, 0))],
        out_specs=pl.BlockSpec((bm, Nn), lambda i: (i, 0)),
        out_shape=jax.ShapeDtypeStruct((M, Nn), jnp.float32),
    )(a, b)


def _head_body(jk_ref, w1_ref, b1_ref, w2_ref, b2_ref, o_ref):
    z = jnp.maximum(
        jnp.dot(jk_ref[...], w1_ref[...],
                preferred_element_type=jnp.float32) + b1_ref[...], 0.0)
    logits = jnp.dot(z, w2_ref[...],
                     preferred_element_type=jnp.float32) + b2_ref[...]
    m = jnp.max(logits, axis=-1, keepdims=True)
    lse = jnp.log(jnp.sum(jnp.exp(logits - m), axis=-1, keepdims=True)) + m
    o_ref[...] = logits - lse


def _head(jk, w1, b1, w2, b2):
    Gr, F = jk.shape
    Hh = w1.shape[1]
    Cp = w2.shape[1]
    return pl.pallas_call(
        _head_body,
        in_specs=[pl.BlockSpec((Gr, F), lambda: (0, 0)),
                  pl.BlockSpec((F, Hh), lambda: (0, 0)),
                  pl.BlockSpec((1, Hh), lambda: (0, 0)),
                  pl.BlockSpec((Hh, Cp), lambda: (0, 0)),
                  pl.BlockSpec((1, Cp), lambda: (0, 0))],
        out_specs=pl.BlockSpec((Gr, Cp), lambda: (0, 0)),
        out_shape=jax.ShapeDtypeStruct((Gr, Cp), jnp.float32),
    )(jk, w1, b1, w2, b2)


def kernel(x, edge_index, batch, w1_rel, b1, w1_root, w2_rel, b2, w2_root,
           w3_rel, b3, w3_root, wp_rel, bp, wp_root, lin1_w, lin1_b,
           lin2_w, lin2_b):
    N, IN_DIM = x.shape
    E = edge_index.shape[1]
    H = w1_rel.shape[1]
    C = lin2_w.shape[1]
    G = 64
    f32 = jnp.float32

    src = edge_index[0].astype(jnp.int32)
    dst = edge_index[1].astype(jnp.int32)
    batch_i = batch.astype(jnp.int32)

    deg = jnp.maximum(
        jax.ops.segment_sum(jnp.ones((E,), jnp.int32), dst, num_segments=N),
        1).astype(f32)
    counts = jax.ops.segment_sum(jnp.ones((N,), jnp.int32), batch_i,
                                 num_segments=G)
    starts = jnp.concatenate(
        [jnp.zeros((1,), jnp.int32), jnp.cumsum(counts)[:-1].astype(jnp.int32)])
    k_arr = jnp.ceil(0.8 * counts.astype(f32)).astype(jnp.int32)
    cum_k = jnp.cumsum(k_arr)
    off_excl = cum_k - k_arr
    K = cum_k[-1]
    intra = jnp.arange(N, dtype=jnp.int32) - starts[batch_i]
    cnt_g = jnp.maximum(counts, 1).astype(f32)
    cnt_new = jnp.maximum(k_arr, 1).astype(f32)

    bm = 256
    Mp = ((N + bm - 1) // bm) * bm
    Kp = ((IN_DIM + 127) // 128) * 128

    x_pad = jnp.zeros((Mp, Kp), f32).at[:N, :IN_DIM].set(x)
    WA = jnp.zeros((Kp, 2 * H), f32)
    WA = WA.at[:IN_DIM, :H].set(w1_rel).at[:IN_DIM, H:].set(w1_root)
    yA = _mm(x_pad, WA)
    agg1 = jax.ops.segment_sum(yA[:N, :H][src], dst,
                               num_segments=N) / deg[:, None]
    h1 = jax.nn.relu(agg1 + b1 + yA[:N, H:])

    h1_pad = jnp.zeros((Mp, H), f32).at[:N].set(h1)
    WB = jnp.concatenate([w2_rel, w2_root], axis=1)
    yB = _mm(h1_pad, WB)
    agg2 = jax.ops.segment_sum(yB[:N, :H][src], dst,
                               num_segments=N) / deg[:, None]
    h2 = jax.nn.relu(agg2 + b2 + yB[:N, H:])

    h2_pad = jnp.zeros((Mp, H), f32).at[:N].set(h2)
    WC = jnp.zeros((H, 128), f32)
    WC = WC.at[:, 0:1].set(wp_rel).at[:, 1:2].set(wp_root)
    yC = _mm(h2_pad, WC)
    s = (jax.ops.segment_sum(yC[:N, 0][src], dst, num_segments=N) / deg
         + bp[0] + yC[:N, 1])

    xs0 = jax.ops.segment_sum(h1, batch_i, num_segments=G) / cnt_g[:, None]
    xs1 = jax.ops.segment_sum(h2, batch_i, num_segments=G) / cnt_g[:, None]

    dense = jnp.full((G, N), -jnp.inf, dtype=f32).at[(batch_i, intra)].set(s)
    order = jnp.argsort(-dense, axis=1)
    j_idx = jnp.arange(N, dtype=jnp.int32)
    valid_j = j_idx < K
    batch_new = jnp.clip(jnp.searchsorted(cum_k, j_idx, side="right"),
                         0, G - 1).astype(jnp.int32)
    rank = j_idx - off_excl[batch_new]
    perm = order[batch_new, rank].astype(jnp.int32) + starts[batch_new]
    perm_safe = jnp.where(valid_j, perm, N)
    xp = h2[perm] * jnp.tanh(s[perm])[:, None]

    kept = jnp.zeros((N + 1,), jnp.bool_).at[perm_safe].set(True)[:N]
    newidx = jnp.zeros((N + 1,), jnp.int32).at[perm_safe].set(j_idx)[:N]
    valid = kept[src] & kept[dst]
    seg_dst = jnp.where(valid, newidx[dst], N)

    xp_pad = jnp.zeros((Mp, H), f32).at[:N].set(xp)
    WD = jnp.concatenate([w3_rel, w3_root], axis=1)
    yD = _mm(xp_pad, WD)
    msg = yD[:N, :H][newidx[src]] * valid[:, None].astype(f32)
    aggp = jax.ops.segment_sum(msg, seg_dst, num_segments=N + 1)[:N]
    degp = jax.ops.segment_sum(valid.astype(f32), seg_dst,
                               num_segments=N + 1)[:N]
    h3 = jax.nn.relu(aggp / jnp.maximum(degp, 1.0)[:, None] + b3
                     + yD[:N, H:])
    seg_new = jnp.where(valid_j, batch_new, G)
    xs2 = (jax.ops.segment_sum(h3, seg_new, num_segments=G + 1)[:G]
           / cnt_new[:, None])

    jk = jnp.concatenate([xs0, xs1, xs2], axis=1)
    Cp = 128
    w2p = jnp.zeros((H, Cp), f32).at[:, :C].set(lin2_w)
    b2p = jnp.full((1, Cp), -1e30, f32).at[0, :C].set(lin2_b)
    out = _head(jk, lin1_w, lin1_b.reshape(1, H), w2p, b2p)
    return out[:, :C]


# speedup vs baseline: 1.2395x; 1.0868x over previous
import jax
import jax.numpy as jnp
from jax.experimental import pallas as pl


def _mm_body(a_ref, b_ref, o_ref):
    o_ref[...] = jnp.dot(a_ref[...], b_ref[...],
                         preferred_element_type=jnp.float32)


def _mm(a, b, bm=256):
    M, K = a.shape
    _, Nn = b.shape
    return pl.pallas_call(
        _mm_body,
        grid=(M // bm,),
        in_specs=[pl.BlockSpec((bm, K), lambda i: (i, 0)),
                  pl.BlockSpec((K, Nn), lambda i: (0, 0))],
        out_specs=pl.BlockSpec((bm, Nn), lambda i: (i, 0)),
        out_shape=jax.ShapeDtypeStruct((M, Nn), jnp.float32),
    )(a, b)


def _head_body(jk_ref, w1_ref, b1_ref, w2_ref, b2_ref, o_ref):
    z = jnp.maximum(
        jnp.dot(jk_ref[...], w1_ref[...],
                preferred_element_type=jnp.float32) + b1_ref[...], 0.0)
    logits = jnp.dot(z, w2_ref[...],
                     preferred_element_type=jnp.float32) + b2_ref[...]
    m = jnp.max(logits, axis=-1, keepdims=True)
    lse = jnp.log(jnp.sum(jnp.exp(logits - m), axis=-1, keepdims=True)) + m
    o_ref[...] = logits - lse


def _head(jk, w1, b1, w2, b2):
    Gr, F = jk.shape
    Hh = w1.shape[1]
    Cp = w2.shape[1]
    return pl.pallas_call(
        _head_body,
        in_specs=[pl.BlockSpec((Gr, F), lambda: (0, 0)),
                  pl.BlockSpec((F, Hh), lambda: (0, 0)),
                  pl.BlockSpec((1, Hh), lambda: (0, 0)),
                  pl.BlockSpec((Hh, Cp), lambda: (0, 0)),
                  pl.BlockSpec((1, Cp), lambda: (0, 0))],
        out_specs=pl.BlockSpec((Gr, Cp), lambda: (0, 0)),
        out_shape=jax.ShapeDtypeStruct((Gr, Cp), jnp.float32),
    )(jk, w1, b1, w2, b2)


def kernel(x, edge_index, batch, w1_rel, b1, w1_root, w2_rel, b2, w2_root,
           w3_rel, b3, w3_root, wp_rel, bp, wp_root, lin1_w, lin1_b,
           lin2_w, lin2_b):
    N, IN_DIM = x.shape
    E = edge_index.shape[1]
    H = w1_rel.shape[1]
    C = lin2_w.shape[1]
    G = 64
    f32 = jnp.float32

    src = edge_index[0].astype(jnp.int32)
    dst = edge_index[1].astype(jnp.int32)
    batch_i = batch.astype(jnp.int32)

    deg = jnp.maximum(
        jax.ops.segment_sum(jnp.ones((E,), jnp.int32), dst, num_segments=N),
        1).astype(f32)
    counts = jax.ops.segment_sum(jnp.ones((N,), jnp.int32), batch_i,
                                 num_segments=G)
    starts = jnp.concatenate(
        [jnp.zeros((1,), jnp.int32), jnp.cumsum(counts)[:-1].astype(jnp.int32)])
    k_arr = jnp.ceil(0.8 * counts.astype(f32)).astype(jnp.int32)
    cum_k = jnp.cumsum(k_arr)
    off_excl = cum_k - k_arr
    K = cum_k[-1]
    intra = jnp.arange(N, dtype=jnp.int32) - starts[batch_i]
    cnt_g = jnp.maximum(counts, 1).astype(f32)
    cnt_new = jnp.maximum(k_arr, 1).astype(f32)

    bm = 256
    Mp = ((N + bm - 1) // bm) * bm
    Kp = ((IN_DIM + 127) // 128) * 128

    x_pad = jnp.zeros((Mp, Kp), f32).at[:N, :IN_DIM].set(x)
    WA = jnp.zeros((Kp, 2 * H), f32)
    WA = WA.at[:IN_DIM, :H].set(w1_rel).at[:IN_DIM, H:].set(w1_root)
    yA = _mm(x_pad, WA)
    agg1 = jax.ops.segment_sum(yA[:N, :H][src], dst,
                               num_segments=N) / deg[:, None]
    h1 = jax.nn.relu(agg1 + b1 + yA[:N, H:])

    h1_pad = jnp.zeros((Mp, H), f32).at[:N].set(h1)
    WB = jnp.concatenate([w2_rel, w2_root], axis=1)
    yB = _mm(h1_pad, WB)
    agg2 = jax.ops.segment_sum(yB[:N, :H][src], dst,
                               num_segments=N) / deg[:, None]
    h2 = jax.nn.relu(agg2 + b2 + yB[:N, H:])

    h2_pad = jnp.zeros((Mp, H), f32).at[:N].set(h2)
    WC = jnp.zeros((H, 128), f32)
    WC = WC.at[:, 0:1].set(wp_rel).at[:, 1:2].set(wp_root)
    yC = _mm(h2_pad, WC)
    s = (jax.ops.segment_sum(yC[:N, 0][src], dst, num_segments=N) / deg
         + bp[0] + yC[:N, 1])

    xs0 = jax.ops.segment_sum(h1, batch_i, num_segments=G) / cnt_g[:, None]
    xs1 = jax.ops.segment_sum(h2, batch_i, num_segments=G) / cnt_g[:, None]

    j_idx = jnp.arange(N, dtype=jnp.int32)
    _, _, sorted_idx = jax.lax.sort((batch_i, -s, j_idx), num_keys=2,
                                    is_stable=True)
    valid_j = j_idx < K
    batch_new = jnp.clip(jnp.searchsorted(cum_k, j_idx, side="right"),
                         0, G - 1).astype(jnp.int32)
    rank = j_idx - off_excl[batch_new]
    perm = sorted_idx[starts[batch_new] + rank]
    perm_safe = jnp.where(valid_j, perm, N)
    xp = h2[perm] * jnp.tanh(s[perm])[:, None]

    kept = jnp.zeros((N + 1,), jnp.bool_).at[perm_safe].set(True)[:N]
    newidx = jnp.zeros((N + 1,), jnp.int32).at[perm_safe].set(j_idx)[:N]
    valid = kept[src] & kept[dst]
    seg_dst = jnp.where(valid, newidx[dst], N)

    xp_pad = jnp.zeros((Mp, H), f32).at[:N].set(xp)
    WD = jnp.concatenate([w3_rel, w3_root], axis=1)
    yD = _mm(xp_pad, WD)
    msg = yD[:N, :H][newidx[src]] * valid[:, None].astype(f32)
    aggp = jax.ops.segment_sum(msg, seg_dst, num_segments=N + 1)[:N]
    degp = jax.ops.segment_sum(valid.astype(f32), seg_dst,
                               num_segments=N + 1)[:N]
    h3 = jax.nn.relu(aggp / jnp.maximum(degp, 1.0)[:, None] + b3
                     + yD[:N, H:])
    seg_new = jnp.where(valid_j, batch_new, G)
    xs2 = (jax.ops.segment_sum(h3, seg_new, num_segments=G + 1)[:G]
           / cnt_new[:, None])

    jk = jnp.concatenate([xs0, xs1, xs2], axis=1)
    Cp = 128
    w2p = jnp.zeros((H, Cp), f32).at[:, :C].set(lin2_w)
    b2p = jnp.full((1, Cp), -1e30, f32).at[0, :C].set(lin2_b)
    out = _head(jk, lin1_w, lin1_b.reshape(1, H), w2p, b2p)
    return out[:, :C]
